# baseline (speedup 1.0000x reference)
import functools

import jax
import jax.numpy as jnp
from jax import lax
from jax.experimental import pallas as pl
from jax.experimental.pallas import tpu as pltpu
from jax.experimental.pallas import tpu_sc as plsc

_B = 1024
_V = 100000
_THR = 0.05
_NC = 2
_NS = 16
_NW = _NC * _NS
_RPW = _B // _NW
_CHUNK = 50000
_NCHUNK = _V // _CHUNK
_NCHAIN = 5
_SUB = _CHUNK // _NCHAIN
_STEPS = _SUB // 16
_BIGI = jnp.int32(2**31 - 1)


def _top2_chunk(buf, goff, chains):
    lane = lax.iota(jnp.int32, 16)
    st = tuple(
        (m1, i1, m2, i2, jnp.int32(goff + j * _SUB) + lane)
        for j, (m1, i1, m2, i2) in enumerate(chains)
    )

    def body(i, st):
        out = []
        for j, (m1, i1, m2, i2, ix) in enumerate(st):
            x = buf[pl.ds(pl.multiple_of(j * _SUB + i * 16, 16), 16)]
            gt1 = x > m1
            gt2 = x > m2
            m2n = jnp.where(gt1, m1, jnp.where(gt2, x, m2))
            i2n = jnp.where(gt1, i1, jnp.where(gt2, ix, i2))
            m1n = jnp.where(gt1, x, m1)
            i1n = jnp.where(gt1, ix, i1)
            out.append((m1n, i1n, m2n, i2n, ix + 16))
        return tuple(out)

    st = plsc.parallel_loop(0, _STEPS, carry=st, unroll=4)(body)
    return tuple((m1, i1, m2, i2) for (m1, i1, m2, i2, _) in st)


def _merge_top2(a, b):
    a1v, a1i, a2v, a2i = a
    b1v, b1i, b2v, b2i = b
    gt = (b1v > a1v) | ((b1v == a1v) & (b1i < a1i))
    m1 = jnp.where(gt, b1v, a1v)
    i1 = jnp.where(gt, b1i, a1i)
    uv = jnp.where(gt, a1v, a2v)
    ui = jnp.where(gt, a1i, a2i)
    wv = jnp.where(gt, b2v, b1v)
    wi = jnp.where(gt, b2i, b1i)
    gt2 = (wv > uv) | ((wv == uv) & (wi < ui))
    m2 = jnp.where(gt2, wv, uv)
    i2 = jnp.where(gt2, wi, ui)
    return (m1, i1, m2, i2)


def _sc_counts(inp, tgt):
    mesh = plsc.VectorSubcoreMesh(core_axis_name="c", subcore_axis_name="s")

    @functools.partial(
        pl.kernel,
        mesh=mesh,
        out_type=jax.ShapeDtypeStruct((_NW, 16), jnp.float32),
        scratch_types=[
            pltpu.VMEM((_CHUNK,), jnp.float32),
            pltpu.VMEM((_CHUNK,), jnp.float32),
            pltpu.VMEM((_RPW,), jnp.int32),
            pltpu.VMEM((16,), jnp.float32),
            pltpu.SemaphoreType.DMA,
            pltpu.SemaphoreType.DMA,
        ],
        compiler_params=pltpu.CompilerParams(
            use_tc_tiling_on_sc=False, needs_layout_passes=False
        ),
    )
    def k(inp_hbm, tgt_hbm, out_hbm, buf0, buf1, tgt_v, out_v, sem0, sem1):
        wid = lax.axis_index("s") * _NC + lax.axis_index("c")
        base = pl.multiple_of(wid * _RPW, _RPW)
        pltpu.sync_copy(tgt_hbm.at[pl.ds(base, _RPW)], tgt_v)

        def start(row, c, buf, sem):
            pltpu.make_async_copy(
                inp_hbm.at[row, pl.ds(c * _CHUNK, _CHUNK)], buf, sem
            ).start()

        def wait(row, c, buf, sem):
            pltpu.make_async_copy(
                inp_hbm.at[row, pl.ds(c * _CHUNK, _CHUNK)], buf, sem
            ).wait()

        start(base, 0, buf0, sem0)
        start(base, 1, buf1, sem1)

        def row_body(r, acc):
            row = base + r
            chains = tuple(
                (
                    jnp.full((16,), -jnp.inf, jnp.float32),
                    jnp.full((16,), _BIGI, jnp.int32),
                    jnp.full((16,), -jnp.inf, jnp.float32),
                    jnp.full((16,), _BIGI, jnp.int32),
                )
                for _ in range(_NCHAIN)
            )
            wait(row, 0, buf0, sem0)
            chains = _top2_chunk(buf0, 0, chains)

            @pl.when(r < _RPW - 1)
            def _():
                start(row + 1, 0, buf0, sem0)

            wait(row, 1, buf1, sem1)
            chains = _top2_chunk(buf1, _CHUNK, chains)

            @pl.when(r < _RPW - 1)
            def _():
                start(row + 1, 1, buf1, sem1)

            st = chains[0]
            for j in range(1, _NCHAIN):
                st = _merge_top2(st, chains[j])
            m1, i1, m2, i2 = st
            M1 = jnp.max(m1)
            eq = m1 == M1
            I1 = jnp.min(jnp.where(eq, i1, _BIGI))
            win = eq & (i1 == I1)
            cv = jnp.where(win, m2, m1)
            ci = jnp.where(win, i2, i1)
            M2 = jnp.max(cv)
            I2 = jnp.min(jnp.where(cv == M2, ci, _BIGI))
            lane = lax.iota(jnp.int32, 16)
            tvec = tgt_v[pl.ds(pl.multiple_of((r // 16) * 16, 16), 16)]
            lsel = lane == (r % 16)
            hit1 = lsel & (tvec == I1)
            hit2 = lsel & (tvec == I2) & (M1 - M2 < _THR)
            return (
                acc
                + jnp.where(hit1, jnp.float32(1.0), jnp.float32(0.0))
                + jnp.where(hit2, jnp.float32(1.0), jnp.float32(0.0))
            )

        acc = lax.fori_loop(
            0, _RPW, row_body, jnp.zeros((16,), jnp.float32)
        )
        out_v[...] = acc
        pltpu.sync_copy(out_v, out_hbm.at[wid])

    return k(inp, tgt)


def _finish(counts):
    def body(x_ref, o_ref):
        o_ref[0] = jnp.float32(1.0) - jnp.sum(x_ref[...]) * jnp.float32(1.0 / _B)

    return pl.pallas_call(
        body,
        out_shape=jax.ShapeDtypeStruct((1,), jnp.float32),
        out_specs=pl.BlockSpec(memory_space=pltpu.SMEM),
    )(counts)


def kernel(input, target):
    counts = _sc_counts(input, target)
    return _finish(counts)[0]


# speedup vs baseline: 1.0375x; 1.0375x over previous
import functools

import jax
import jax.numpy as jnp
from jax import lax
from jax.experimental import pallas as pl
from jax.experimental.pallas import tpu as pltpu
from jax.experimental.pallas import tpu_sc as plsc

_B = 1024
_V = 100000
_THR = 0.05
_NC = 2
_NS = 16
_NW = _NC * _NS
_RPW = _B // _NW
_CHUNK = 50000
_NCHUNK = _V // _CHUNK
_NCH = 5
_CHSPAN = _CHUNK // _NCH
_BLKV = 25
_BLK = _BLKV * 16
_NBLK = _CHSPAN // _BLK
_RCH = 5
_RSPAN = _BLK // _RCH
_RV = _RSPAN // 16
_BIGI = jnp.int32(2**31 - 1)


def _merge_top2(a, b):
    a1v, a1i, a2v, a2i = a
    b1v, b1i, b2v, b2i = b
    gt = (b1v > a1v) | ((b1v == a1v) & (b1i < a1i))
    m1 = jnp.where(gt, b1v, a1v)
    i1 = jnp.where(gt, b1i, a1i)
    uv = jnp.where(gt, a1v, a2v)
    ui = jnp.where(gt, a1i, a2i)
    wv = jnp.where(gt, b2v, b1v)
    wi = jnp.where(gt, b2i, b1i)
    gt2 = (wv > uv) | ((wv == uv) & (wi < ui))
    m2 = jnp.where(gt2, wv, uv)
    i2 = jnp.where(gt2, wi, ui)
    return (m1, i1, m2, i2)


def _chunk_top2(buf, chunk_off, state, lane):
    ninf = jnp.full((16,), -jnp.inf, jnp.float32)
    bigi = jnp.full((16,), _BIGI, jnp.int32)
    init = tuple((ninf, bigi, ninf, bigi) for _ in range(_NCH))

    def body(blk, st):
        out = []
        boff = blk * _BLK
        for j, (bm1, bb1, bm2, bb2) in enumerate(st):
            base = j * _CHSPAN + boff
            mv = buf[pl.ds(pl.multiple_of(base, 16), 16)]
            for v in range(1, _BLKV):
                x = buf[pl.ds(pl.multiple_of(base + v * 16, 16), 16)]
                mv = jnp.maximum(mv, x)
            bid = jnp.int32(j * _CHSPAN) + boff
            gt1 = mv > bm1
            gt2 = mv > bm2
            bm2n = jnp.where(gt1, bm1, jnp.where(gt2, mv, bm2))
            bb2n = jnp.where(gt1, bb1, jnp.where(gt2, bid, bb2))
            bm1n = jnp.where(gt1, mv, bm1)
            bb1n = jnp.where(gt1, bid, bb1)
            out.append((bm1n, bb1n, bm2n, bb2n))
        return tuple(out)

    st = plsc.parallel_loop(0, _NBLK, carry=init, unroll=1)(body)
    bt = st[0]
    for j in range(1, _NCH):
        bt = _merge_top2(bt, st[j])
    bm1, bb1, bm2, bb2 = bt

    V1 = jnp.max(bm1)
    eqv = bm1 == V1
    B1 = jnp.min(jnp.where(eqv, bb1, _BIGI))
    win = eqv & (bb1 == B1)
    cv = jnp.where(win, bm2, bm1)
    ci = jnp.where(win, bb2, bb1)
    V2 = jnp.max(cv)
    B2 = jnp.min(jnp.where(cv == V2, ci, _BIGI))

    def rescan(bb):
        chains = []
        for k in range(_RCH):
            m1 = ninf
            i1 = bigi
            m2 = ninf
            i2 = bigi
            for v in range(_RV):
                lidx = bb + (k * _RSPAN + v * 16) + lane
                x = plsc.load_gather(buf, [lidx])
                gidx = jnp.int32(chunk_off) + lidx
                gt1 = x > m1
                gt2 = x > m2
                m2 = jnp.where(gt1, m1, jnp.where(gt2, x, m2))
                i2 = jnp.where(gt1, i1, jnp.where(gt2, gidx, i2))
                m1 = jnp.where(gt1, x, m1)
                i1 = jnp.where(gt1, gidx, i1)
            chains.append((m1, i1, m2, i2))
        s = chains[0]
        for k in range(1, _RCH):
            s = _merge_top2(s, chains[k])
        return s

    s1 = rescan(B1)
    s2 = rescan(B2)
    sm = _merge_top2(s1, s2)
    same = B1 == B2
    sc = tuple(jnp.where(same, a, b) for a, b in zip(s1, sm))
    return _merge_top2(state, sc)


def _sc_counts(inp, tgt):
    mesh = plsc.VectorSubcoreMesh(core_axis_name="c", subcore_axis_name="s")

    @functools.partial(
        pl.kernel,
        mesh=mesh,
        out_type=jax.ShapeDtypeStruct((_NW, 16), jnp.float32),
        scratch_types=[
            pltpu.VMEM((_CHUNK,), jnp.float32),
            pltpu.VMEM((_CHUNK,), jnp.float32),
            pltpu.VMEM((_RPW,), jnp.int32),
            pltpu.VMEM((16,), jnp.float32),
            pltpu.SemaphoreType.DMA,
            pltpu.SemaphoreType.DMA,
        ],
        compiler_params=pltpu.CompilerParams(
            use_tc_tiling_on_sc=False, needs_layout_passes=False
        ),
    )
    def k(inp_hbm, tgt_hbm, out_hbm, buf0, buf1, tgt_v, out_v, sem0, sem1):
        wid = lax.axis_index("s") * _NC + lax.axis_index("c")
        base = pl.multiple_of(wid * _RPW, _RPW)
        pltpu.sync_copy(tgt_hbm.at[pl.ds(base, _RPW)], tgt_v)
        lane = lax.iota(jnp.int32, 16)

        def start(row, c, buf, sem):
            pltpu.make_async_copy(
                inp_hbm.at[row, pl.ds(c * _CHUNK, _CHUNK)], buf, sem
            ).start()

        def wait(row, c, buf, sem):
            pltpu.make_async_copy(
                inp_hbm.at[row, pl.ds(c * _CHUNK, _CHUNK)], buf, sem
            ).wait()

        start(base, 0, buf0, sem0)
        start(base, 1, buf1, sem1)

        def row_body(r, acc):
            row = base + r
            ninf = jnp.full((16,), -jnp.inf, jnp.float32)
            bigi = jnp.full((16,), _BIGI, jnp.int32)
            state = (ninf, bigi, ninf, bigi)

            wait(row, 0, buf0, sem0)
            state = _chunk_top2(buf0, 0, state, lane)

            @pl.when(r < _RPW - 1)
            def _():
                start(row + 1, 0, buf0, sem0)

            wait(row, 1, buf1, sem1)
            state = _chunk_top2(buf1, _CHUNK, state, lane)

            @pl.when(r < _RPW - 1)
            def _():
                start(row + 1, 1, buf1, sem1)

            m1, i1, m2, i2 = state
            M1 = jnp.max(m1)
            eq = m1 == M1
            I1 = jnp.min(jnp.where(eq, i1, _BIGI))
            win = eq & (i1 == I1)
            cv = jnp.where(win, m2, m1)
            ci = jnp.where(win, i2, i1)
            M2 = jnp.max(cv)
            I2 = jnp.min(jnp.where(cv == M2, ci, _BIGI))
            tvec = tgt_v[pl.ds(pl.multiple_of((r // 16) * 16, 16), 16)]
            lsel = lane == (r % 16)
            hit1 = lsel & (tvec == I1)
            hit2 = lsel & (tvec == I2) & (M1 - M2 < _THR)
            return (
                acc
                + jnp.where(hit1, jnp.float32(1.0), jnp.float32(0.0))
                + jnp.where(hit2, jnp.float32(1.0), jnp.float32(0.0))
            )

        acc = lax.fori_loop(
            0, _RPW, row_body, jnp.zeros((16,), jnp.float32)
        )
        out_v[...] = acc
        pltpu.sync_copy(out_v, out_hbm.at[wid])

    return k(inp, tgt)


def _finish(counts):
    def body(x_ref, o_ref):
        o_ref[0] = jnp.float32(1.0) - jnp.sum(x_ref[...]) * jnp.float32(1.0 / _B)

    return pl.pallas_call(
        body,
        out_shape=jax.ShapeDtypeStruct((1,), jnp.float32),
        out_specs=pl.BlockSpec(memory_space=pltpu.SMEM),
    )(counts)


def kernel(input, target):
    counts = _sc_counts(input, target)
    return _finish(counts)[0]


# speedup vs baseline: 1.0770x; 1.0381x over previous
import functools

import jax
import jax.numpy as jnp
from jax import lax
from jax.experimental import pallas as pl
from jax.experimental.pallas import tpu as pltpu
from jax.experimental.pallas import tpu_sc as plsc

_B = 1024
_V = 100000
_THR = 0.05
_NC = 2
_NS = 16
_NW = _NC * _NS
_RPW = _B // _NW
_CHUNK = 20000
_NCHUNK = _V // _CHUNK
_NCH = 5
_CHSPAN = _CHUNK // _NCH
_BLKV = 25
_BLK = _BLKV * 16
_NBLK = _CHSPAN // _BLK
_RCH = 5
_RSPAN = _BLK // _RCH
_RV = _RSPAN // 16
_BIGI = jnp.int32(2**31 - 1)


def _merge_top2(a, b):
    a1v, a1i, a2v, a2i = a
    b1v, b1i, b2v, b2i = b
    gt = (b1v > a1v) | ((b1v == a1v) & (b1i < a1i))
    m1 = jnp.where(gt, b1v, a1v)
    i1 = jnp.where(gt, b1i, a1i)
    uv = jnp.where(gt, a1v, a2v)
    ui = jnp.where(gt, a1i, a2i)
    wv = jnp.where(gt, b2v, b1v)
    wi = jnp.where(gt, b2i, b1i)
    gt2 = (wv > uv) | ((wv == uv) & (wi < ui))
    m2 = jnp.where(gt2, wv, uv)
    i2 = jnp.where(gt2, wi, ui)
    return (m1, i1, m2, i2)


def _chunk_top2(buf, chunk_off, state, lane):
    ninf = jnp.full((16,), -jnp.inf, jnp.float32)
    bigi = jnp.full((16,), _BIGI, jnp.int32)
    init = tuple((ninf, bigi, ninf, bigi) for _ in range(_NCH))

    def body(blk, st):
        out = []
        boff = blk * _BLK
        for j, (bm1, bb1, bm2, bb2) in enumerate(st):
            base = j * _CHSPAN + boff
            mv = buf[pl.ds(pl.multiple_of(base, 16), 16)]
            for v in range(1, _BLKV):
                x = buf[pl.ds(pl.multiple_of(base + v * 16, 16), 16)]
                mv = jnp.maximum(mv, x)
            bid = jnp.int32(j * _CHSPAN) + boff
            gt1 = mv > bm1
            gt2 = mv > bm2
            bm2n = jnp.where(gt1, bm1, jnp.where(gt2, mv, bm2))
            bb2n = jnp.where(gt1, bb1, jnp.where(gt2, bid, bb2))
            bm1n = jnp.where(gt1, mv, bm1)
            bb1n = jnp.where(gt1, bid, bb1)
            out.append((bm1n, bb1n, bm2n, bb2n))
        return tuple(out)

    st = plsc.parallel_loop(0, _NBLK, carry=init, unroll=1)(body)
    bt = st[0]
    for j in range(1, _NCH):
        bt = _merge_top2(bt, st[j])
    bm1, bb1, bm2, bb2 = bt

    V1 = jnp.max(bm1)
    eqv = bm1 == V1
    B1 = jnp.min(jnp.where(eqv, bb1, _BIGI))
    win = eqv & (bb1 == B1)
    cv = jnp.where(win, bm2, bm1)
    ci = jnp.where(win, bb2, bb1)
    V2 = jnp.max(cv)
    B2 = jnp.min(jnp.where(cv == V2, ci, _BIGI))

    def rescan(bb):
        chains = []
        for k in range(_RCH):
            m1 = ninf
            i1 = bigi
            m2 = ninf
            i2 = bigi
            for v in range(_RV):
                lidx = bb + (k * _RSPAN + v * 16) + lane
                x = plsc.load_gather(buf, [lidx])
                gidx = jnp.int32(chunk_off) + lidx
                gt1 = x > m1
                gt2 = x > m2
                m2 = jnp.where(gt1, m1, jnp.where(gt2, x, m2))
                i2 = jnp.where(gt1, i1, jnp.where(gt2, gidx, i2))
                m1 = jnp.where(gt1, x, m1)
                i1 = jnp.where(gt1, gidx, i1)
            chains.append((m1, i1, m2, i2))
        s = chains[0]
        for k in range(1, _RCH):
            s = _merge_top2(s, chains[k])
        return s

    s1 = rescan(B1)
    s2 = rescan(B2)
    sm = _merge_top2(s1, s2)
    same = B1 == B2
    sc = tuple(jnp.where(same, a, b) for a, b in zip(s1, sm))
    return _merge_top2(state, sc)


def _sc_counts(inp, tgt):
    mesh = plsc.VectorSubcoreMesh(core_axis_name="c", subcore_axis_name="s")

    @functools.partial(
        pl.kernel,
        mesh=mesh,
        out_type=jax.ShapeDtypeStruct((_NW, 16), jnp.float32),
        scratch_types=(
            [pltpu.VMEM((_CHUNK,), jnp.float32) for _ in range(_NCHUNK)]
            + [
                pltpu.VMEM((_RPW,), jnp.int32),
                pltpu.VMEM((16,), jnp.float32),
            ]
            + [pltpu.SemaphoreType.DMA for _ in range(_NCHUNK)]
        ),
        compiler_params=pltpu.CompilerParams(
            use_tc_tiling_on_sc=False, needs_layout_passes=False
        ),
    )
    def k(inp_hbm, tgt_hbm, out_hbm, *scratch):
        bufs = scratch[:_NCHUNK]
        tgt_v = scratch[_NCHUNK]
        out_v = scratch[_NCHUNK + 1]
        sems = scratch[_NCHUNK + 2:]
        wid = lax.axis_index("s") * _NC + lax.axis_index("c")
        base = pl.multiple_of(wid * _RPW, _RPW)
        pltpu.sync_copy(tgt_hbm.at[pl.ds(base, _RPW)], tgt_v)
        lane = lax.iota(jnp.int32, 16)

        def start(row, c, buf, sem):
            pltpu.make_async_copy(
                inp_hbm.at[row, pl.ds(c * _CHUNK, _CHUNK)], buf, sem
            ).start()

        def wait(row, c, buf, sem):
            pltpu.make_async_copy(
                inp_hbm.at[row, pl.ds(c * _CHUNK, _CHUNK)], buf, sem
            ).wait()

        for c in range(_NCHUNK):
            start(base, c, bufs[c], sems[c])

        def row_body(r, acc):
            row = base + r
            ninf = jnp.full((16,), -jnp.inf, jnp.float32)
            bigi = jnp.full((16,), _BIGI, jnp.int32)
            state = (ninf, bigi, ninf, bigi)

            for c in range(_NCHUNK):
                wait(row, c, bufs[c], sems[c])
                state = _chunk_top2(bufs[c], c * _CHUNK, state, lane)

                @pl.when(r < _RPW - 1)
                def _(c=c):
                    start(row + 1, c, bufs[c], sems[c])

            m1, i1, m2, i2 = state
            M1 = jnp.max(m1)
            eq = m1 == M1
            I1 = jnp.min(jnp.where(eq, i1, _BIGI))
            win = eq & (i1 == I1)
            cv = jnp.where(win, m2, m1)
            ci = jnp.where(win, i2, i1)
            M2 = jnp.max(cv)
            I2 = jnp.min(jnp.where(cv == M2, ci, _BIGI))
            tvec = tgt_v[pl.ds(pl.multiple_of((r // 16) * 16, 16), 16)]
            lsel = lane == (r % 16)
            hit1 = lsel & (tvec == I1)
            hit2 = lsel & (tvec == I2) & (M1 - M2 < _THR)
            return (
                acc
                + jnp.where(hit1, jnp.float32(1.0), jnp.float32(0.0))
                + jnp.where(hit2, jnp.float32(1.0), jnp.float32(0.0))
            )

        acc = lax.fori_loop(
            0, _RPW, row_body, jnp.zeros((16,), jnp.float32)
        )
        out_v[...] = acc
        pltpu.sync_copy(out_v, out_hbm.at[wid])

    return k(inp, tgt)


def _finish(counts):
    def body(x_ref, o_ref):
        o_ref[0] = jnp.float32(1.0) - jnp.sum(x_ref[...]) * jnp.float32(1.0 / _B)

    return pl.pallas_call(
        body,
        out_shape=jax.ShapeDtypeStruct((1,), jnp.float32),
        out_specs=pl.BlockSpec(memory_space=pltpu.SMEM),
    )(counts)


def kernel(input, target):
    counts = _sc_counts(input, target)
    return _finish(counts)[0]
